# baseline (speedup 1.0000x reference)
import functools

import jax
import jax.numpy as jnp
from jax import lax
from jax.experimental import pallas as pl
from jax.experimental.pallas import tpu as pltpu
from jax.experimental.pallas import tpu_sc as plsc

N = 50000
E = 800000
D = 81
DP = 128
NP = 50176
NB = 512
RNG = 12512
SLAB = RNG + 32
CHUNK = 112
E2 = 806400
CPT = E2 // CHUNK // 16
BE = 3200


def _edge_emb_body(attr_ref, w_ref, b_ref, out_ref):
    out_ref[...] = (
        jnp.dot(attr_ref[...], w_ref[...], preferred_element_type=jnp.float32)
        + b_ref[...]
    )


def _edge_emb(attr_p, w_l, b_l):
    return pl.pallas_call(
        _edge_emb_body,
        grid=(E2 // BE,),
        in_specs=[
            pl.BlockSpec((BE, 8), lambda i: (i, 0)),
            pl.BlockSpec((8, DP), lambda i: (0, 0)),
            pl.BlockSpec((1, DP), lambda i: (0, 0)),
        ],
        out_specs=pl.BlockSpec((BE, DP), lambda i: (i, 0)),
        out_shape=jax.ShapeDtypeStruct((E2, DP), jnp.float32),
    )(attr_p, w_l, b_l)


def _mlp1_body(eps_ref, h_ref, agg_ref, w1_ref, b1_ref, t_ref, stats_ref,
               acc_s, acc_q):
    i = pl.program_id(0)
    z = (1.0 + eps_ref[0]) * h_ref[...] + agg_ref[...]
    t = jnp.dot(z, w1_ref[...], preferred_element_type=jnp.float32) + b1_ref[...]
    t_ref[...] = t
    rows = i * NB + lax.broadcasted_iota(jnp.int32, (NB, 1), 0)
    tm = jnp.where(rows < N, t, 0.0)

    @pl.when(i == 0)
    def _():
        acc_s[...] = jnp.zeros_like(acc_s)
        acc_q[...] = jnp.zeros_like(acc_q)

    acc_s[...] += jnp.sum(tm, axis=0, keepdims=True)
    acc_q[...] += jnp.sum(tm * tm, axis=0, keepdims=True)

    @pl.when(i == pl.num_programs(0) - 1)
    def _():
        mu = acc_s[...] / N
        var = acc_q[...] / N - mu * mu
        stats_ref[0:1, :] = mu
        stats_ref[1:2, :] = var


def _mlp1(h, agg, w1_l, b1_l, eps_l):
    return pl.pallas_call(
        _mlp1_body,
        grid=(NP // NB,),
        in_specs=[
            pl.BlockSpec(memory_space=pltpu.SMEM),
            pl.BlockSpec((NB, DP), lambda i: (i, 0)),
            pl.BlockSpec((NB, DP), lambda i: (i, 0)),
            pl.BlockSpec((DP, DP), lambda i: (0, 0)),
            pl.BlockSpec((1, DP), lambda i: (0, 0)),
        ],
        out_specs=[
            pl.BlockSpec((NB, DP), lambda i: (i, 0)),
            pl.BlockSpec((2, DP), lambda i: (0, 0)),
        ],
        out_shape=[
            jax.ShapeDtypeStruct((NP, DP), jnp.float32),
            jax.ShapeDtypeStruct((2, DP), jnp.float32),
        ],
        scratch_shapes=[
            pltpu.VMEM((1, DP), jnp.float32),
            pltpu.VMEM((1, DP), jnp.float32),
        ],
    )(eps_l, h, agg, w1_l, b1_l)


def _mlp2_body(t_ref, stats_ref, g1_ref, bn1_ref, w2_ref, b2_ref, geff_ref,
               beta_ref, hin_ref, out_ref, *, last):
    mu = stats_ref[0:1, :]
    var = stats_ref[1:2, :]
    tn = (t_ref[...] - mu) * lax.rsqrt(var + 1e-5) * g1_ref[...] + bn1_ref[...]
    tn = jnp.maximum(tn, 0.0)
    u = jnp.dot(tn, w2_ref[...], preferred_element_type=jnp.float32) + b2_ref[...]
    u = jnp.maximum(u, 0.0)
    hb = u * geff_ref[...] + beta_ref[...]
    if not last:
        hb = jnp.maximum(hb, 0.0)
    out_ref[...] = hb + hin_ref[...]


def _mlp2(t, stats, g1_l, bn1_l, w2_l, b2_l, geff_l, beta_l, hin, last):
    return pl.pallas_call(
        functools.partial(_mlp2_body, last=last),
        grid=(NP // NB,),
        in_specs=[
            pl.BlockSpec((NB, DP), lambda i: (i, 0)),
            pl.BlockSpec((2, DP), lambda i: (0, 0)),
            pl.BlockSpec((1, DP), lambda i: (0, 0)),
            pl.BlockSpec((1, DP), lambda i: (0, 0)),
            pl.BlockSpec((DP, DP), lambda i: (0, 0)),
            pl.BlockSpec((1, DP), lambda i: (0, 0)),
            pl.BlockSpec((1, DP), lambda i: (0, 0)),
            pl.BlockSpec((1, DP), lambda i: (0, 0)),
            pl.BlockSpec((NB, DP), lambda i: (i, 0)),
        ],
        out_specs=pl.BlockSpec((NB, DP), lambda i: (i, 0)),
        out_shape=jax.ShapeDtypeStruct((NP, DP), jnp.float32),
    )(t, stats, g1_l, bn1_l, w2_l, b2_l, geff_l, beta_l, hin)


def _edge_agg_body(h_hbm, e_hbm, src_hbm, dst_hbm, agg_hbm,
                   svm, dvm, rvm, evm, hvm, slab, sem):
    c = lax.axis_index("c")
    s = lax.axis_index("s")
    lane = lax.iota(jnp.int32, 16)

    for rb in range(2):
        b = 2 * c + rb
        lo = b * RNG

        def _zrow(i, _):
            for k in range(DP // 16):
                evm[i, pl.ds(k * 16, 16)] = jnp.zeros((16,), jnp.float32)
            return _
        lax.fori_loop(0, CHUNK, _zrow, None)

        def _zslab(k, _):
            pltpu.sync_copy(evm, slab.at[pl.ds(s * 784 + k * 112, 112)])
            return _
        lax.fori_loop(0, 7, _zslab, None)
        plsc.subcore_barrier()

        def _chunk(j, _):
            base = (s * CPT + j) * CHUNK
            pltpu.sync_copy(src_hbm.at[pl.ds(base, CHUNK)], svm)
            pltpu.sync_copy(dst_hbm.at[pl.ds(base, CHUNK)], dvm)
            pltpu.sync_copy(e_hbm.at[pl.ds(base, CHUNK)], evm)

            pltpu.async_copy(h_hbm.at[svm], hvm, sem).wait()

            def _rel(g, _):
                dv = dvm[pl.ds(g * 16, 16)]
                inr = (dv >= lo) & (dv < lo + RNG)
                rel = jnp.where(inr, dv - lo,
                                RNG + lane + 16 * (g % 2))
                rvm[pl.ds(g * 16, 16)] = rel
                return _
            lax.fori_loop(0, CHUNK // 16, _rel, None)

            def _msg(m, _):
                for k in range(DP // 16):
                    hv = hvm[m, pl.ds(k * 16, 16)]
                    ev = evm[m, pl.ds(k * 16, 16)]
                    hvm[m, pl.ds(k * 16, 16)] = jnp.maximum(hv + ev, 0.0)
                return _
            lax.fori_loop(0, CHUNK, _msg, None)

            pltpu.sync_copy(hvm, slab.at[rvm], add=True)
            return _
        lax.fori_loop(0, CPT, _chunk, None)
        plsc.subcore_barrier()

        @pl.when(s < 15)
        def _():
            pltpu.sync_copy(slab.at[pl.ds(s * 784, 784)],
                            agg_hbm.at[pl.ds(lo + s * 784, 784)])

        @pl.when(s == 15)
        def _():
            pltpu.sync_copy(slab.at[pl.ds(15 * 784, 752)],
                            agg_hbm.at[pl.ds(lo + 15 * 784, 752)])
        plsc.subcore_barrier()


def _edge_agg(h, e, src, dst):
    mesh = plsc.VectorSubcoreMesh(core_axis_name="c", subcore_axis_name="s")
    f = functools.partial(
        pl.kernel,
        mesh=mesh,
        out_type=jax.ShapeDtypeStruct((NP, DP), jnp.float32),
        scratch_types=[
            pltpu.VMEM((CHUNK,), jnp.int32),
            pltpu.VMEM((CHUNK,), jnp.int32),
            pltpu.VMEM((CHUNK,), jnp.int32),
            pltpu.VMEM((CHUNK, DP), jnp.float32),
            pltpu.VMEM((CHUNK, DP), jnp.float32),
            pltpu.VMEM_SHARED((SLAB, DP), jnp.float32),
            pltpu.SemaphoreType.DMA,
        ],
    )(_edge_agg_body)
    return f(h, e, src, dst)


def kernel(x, edge_index, edge_attr, We, be, W1, b1, g1, bn1, W2, b2, eps,
           gamma, beta):
    f32 = jnp.float32
    h = jnp.pad(x.astype(f32), ((0, NP - N), (0, DP - D)))
    src = jnp.pad(edge_index[0], (0, E2 - E))
    dst = jnp.pad(edge_index[1], (0, E2 - E), constant_values=NP)
    attr_p = jnp.pad(edge_attr, ((0, E2 - E), (0, 2)))
    We_p = jnp.pad(We, ((0, 0), (0, 2), (0, DP - D)))
    be_p = jnp.pad(be, ((0, 0), (0, DP - D)))
    W1_p = jnp.pad(W1, ((0, 0), (0, DP - D), (0, DP - D)))
    b1_p = jnp.pad(b1, ((0, 0), (0, DP - D)))
    g1_p = jnp.pad(g1, ((0, 0), (0, DP - D)))
    bn1_p = jnp.pad(bn1, ((0, 0), (0, DP - D)))
    W2_p = jnp.pad(W2, ((0, 0), (0, DP - D), (0, DP - D)))
    b2_p = jnp.pad(b2, ((0, 0), (0, DP - D)))
    geff = jnp.pad(gamma / jnp.sqrt(1.0 + 1e-5), ((0, 0), (0, DP - D)))
    beta_p = jnp.pad(beta, ((0, 0), (0, DP - D)))

    L = We.shape[0]
    for l in range(L):
        e = _edge_emb(attr_p, We_p[l], be_p[l].reshape(1, DP))
        agg = _edge_agg(h, e, src, dst)
        t, stats = _mlp1(h, agg, W1_p[l], b1_p[l].reshape(1, DP),
                         eps[l].reshape(1))
        h = _mlp2(t, stats, g1_p[l].reshape(1, DP), bn1_p[l].reshape(1, DP),
                  W2_p[l], b2_p[l].reshape(1, DP), geff[l].reshape(1, DP),
                  beta_p[l].reshape(1, DP), h, last=(l == L - 1))
    return h[:N, :D]
